# baseline (speedup 1.0000x reference)
import jax
import jax.numpy as jnp
import numpy as np
from jax import lax
from jax.experimental import pallas as pl
from jax.experimental.pallas import tpu as pltpu
from jax.experimental.pallas import tpu_sc as plsc

N = 100000
E = 200000
D_NODE = 128
D_EDGE = 16
D_OUT = 128
DEG = 4

NC = 2
NS = 16
NW = NC * NS

C = 32
NCHUNK = N // C
TMAX = (NCHUNK + NW - 1) // NW
NREM = NCHUNK - (TMAX - 1) * NW
NPAD = NW * TMAX * C

EBN = 2048
ESTEPS = (E + EBN - 1) // EBN
EPAD = ESTEPS * EBN

_U_PLACE = np.zeros((D_EDGE, 8, 128), np.float32)
_U_PLACE[np.arange(D_EDGE)[:, None], np.arange(8)[None, :],
         16 * np.arange(8)[None, :] + np.arange(D_EDGE)[:, None]] = 1.0


def _prep_bond_kernel(btab_ref, u_ref, bond2_ref):
    del u_ref
    y = jnp.transpose(btab_ref[...])
    y3 = y.reshape(EBN // 8, 8, D_EDGE)
    bond2_ref[...] = jnp.concatenate([y3[:, pp, :] for pp in range(8)],
                                     axis=1)


@jax.jit
def _prep_bond(btab):
    return pl.pallas_call(
        _prep_bond_kernel,
        grid=(ESTEPS,),
        in_specs=[
            pl.BlockSpec((D_EDGE, EBN), lambda i: (0, i)),
            pl.BlockSpec((D_EDGE, 8, 128), lambda i: (0, 0, 0)),
        ],
        out_specs=pl.BlockSpec((EBN // 8, 128), lambda i: (i, 0)),
        out_shape=jax.ShapeDtypeStruct((EPAD // 8, 128), jnp.float32),
    )(btab, _U_PLACE)


def _sc_atom_kernel(atom_hbm, at_hbm, nbr_out,
                    raw_i, idx_all, bufa0, bufa1, acca0, acca1,
                    sga0, sga1, swa0, swa1):
    wid = lax.axis_index("c") * NS + lax.axis_index("s")
    start_c = wid * (TMAX - 1) + jnp.minimum(wid, NREM)
    nt = jnp.where(wid < NREM, TMAX, TMAX - 1)
    a0 = start_c * C

    pltpu.sync_copy(at_hbm.at[pl.ds(1, DEG), pl.ds(a0, TMAX * C)], raw_i)

    def build(t, _):
        for k in range(DEG):
            for h in range(C // 16):
                idx_all[t, pl.ds(C * k + 16 * h, 16)] = (
                    raw_i[k, pl.ds(C * t + 16 * h, 16)])
        return 0

    lax.fori_loop(0, TMAX, build, 0)

    def issue_gather(t, bufa, sa):
        pltpu.async_copy(atom_hbm.at[idx_all.at[t]], bufa, sa)

    def wait_gather(t, bufa, sa):
        pltpu.make_async_copy(atom_hbm.at[idx_all.at[t]], bufa, sa).wait()

    def do_sums(bufa, acca):
        @plsc.parallel_loop(0, C, 1, unroll=4)
        def _(r):
            for d in range(D_NODE // 16):
                sl = pl.ds(16 * d, 16)
                acca[r, sl] = (bufa[r, sl] + bufa[C + r, sl]
                               + bufa[2 * C + r, sl] + bufa[3 * C + r, sl])

    def issue_writes(t, acca, sa):
        pltpu.async_copy(acca, nbr_out.at[pl.ds((start_c + t) * C, C)], sa)

    def wait_writes(acca, sa):
        pltpu.make_async_copy(acca, nbr_out.at[pl.ds(0, C)], sa).wait()

    issue_gather(0, bufa0, sga0)

    def body(u, _):
        t0 = 2 * u
        t1 = t0 + 1

        @pl.when(t1 < nt)
        def _():
            issue_gather(t1, bufa1, sga1)

        wait_gather(t0, bufa0, sga0)

        @pl.when(u >= 1)
        def _():
            wait_writes(acca0, swa0)

        do_sums(bufa0, acca0)
        issue_writes(t0, acca0, swa0)

        @pl.when(t1 < nt)
        def _():
            @pl.when(t1 + 1 < nt)
            def _():
                issue_gather(t1 + 1, bufa0, sga0)

            wait_gather(t1, bufa1, sga1)

            @pl.when(u >= 1)
            def _():
                wait_writes(acca1, swa1)

            do_sums(bufa1, acca1)
            issue_writes(t1, acca1, swa1)

        return 0

    lax.fori_loop(0, (nt + 1) // 2, body, 0)
    wait_writes(acca0, swa0)
    wait_writes(acca1, swa1)


@jax.jit
def _sc_atom(atom_repr, at_idx):
    mesh = plsc.VectorSubcoreMesh(core_axis_name="c", subcore_axis_name="s")
    f = pl.kernel(
        _sc_atom_kernel,
        out_type=jax.ShapeDtypeStruct((N, D_NODE), jnp.float32),
        mesh=mesh,
        scratch_types=[
            pltpu.VMEM((DEG, TMAX * C), jnp.int32),
            pltpu.VMEM((TMAX, 4 * C), jnp.int32),
            pltpu.VMEM((4 * C, D_NODE), jnp.float32),
            pltpu.VMEM((4 * C, D_NODE), jnp.float32),
            pltpu.VMEM((C, D_NODE), jnp.float32),
            pltpu.VMEM((C, D_NODE), jnp.float32),
        ] + [pltpu.SemaphoreType.DMA] * 4,
        compiler_params=pltpu.CompilerParams(use_tc_tiling_on_sc=False),
    )
    return f(atom_repr, at_idx)


def _sc_bond_kernel(bond_hbm, bt_hbm, bnd_out,
                    raw_i, idx_all, bufb0, bufb1, accb0, accb1,
                    sgb0, sgb1, swb0, swb1):
    wid = lax.axis_index("c") * NS + lax.axis_index("s")
    start_c = wid * (TMAX - 1) + jnp.minimum(wid, NREM)
    nt = jnp.where(wid < NREM, TMAX, TMAX - 1)
    a0 = start_c * C

    pltpu.sync_copy(bt_hbm.at[:, pl.ds(a0, TMAX * C)], raw_i)

    def build(t, _):
        for k in range(DEG):
            for h in range(C // 16):
                idx_all[t, pl.ds(C * k + 16 * h, 16)] = (
                    raw_i[k, pl.ds(C * t + 16 * h, 16)])
        return 0

    lax.fori_loop(0, TMAX, build, 0)

    def issue_gather(t, bufb, sb):
        pltpu.async_copy(bond_hbm.at[idx_all.at[t]], bufb, sb)

    def wait_gather(t, bufb, sb):
        pltpu.make_async_copy(bond_hbm.at[idx_all.at[t]], bufb, sb).wait()

    def do_sums(bufb, accb):
        @plsc.parallel_loop(0, C, 1, unroll=4)
        def _(r):
            accb[r, :] = (bufb[r, :] + bufb[C + r, :]
                          + bufb[2 * C + r, :] + bufb[3 * C + r, :])

    def issue_writes(t, accb, sb):
        obase = (start_c + t) * C
        pltpu.async_copy(accb, bnd_out.at[pl.ds(obase, C), pl.ds(0, D_EDGE)],
                         sb)

    def wait_writes(accb, sb):
        pltpu.make_async_copy(accb, bnd_out.at[pl.ds(0, C), pl.ds(0, D_EDGE)],
                              sb).wait()

    issue_gather(0, bufb0, sgb0)

    def body(u, _):
        t0 = 2 * u
        t1 = t0 + 1

        @pl.when(t1 < nt)
        def _():
            issue_gather(t1, bufb1, sgb1)

        wait_gather(t0, bufb0, sgb0)

        @pl.when(u >= 1)
        def _():
            wait_writes(accb0, swb0)

        do_sums(bufb0, accb0)
        issue_writes(t0, accb0, swb0)

        @pl.when(t1 < nt)
        def _():
            @pl.when(t1 + 1 < nt)
            def _():
                issue_gather(t1 + 1, bufb0, sgb0)

            wait_gather(t1, bufb1, sgb1)

            @pl.when(u >= 1)
            def _():
                wait_writes(accb1, swb1)

            do_sums(bufb1, accb1)
            issue_writes(t1, accb1, swb1)

        return 0

    lax.fori_loop(0, (nt + 1) // 2, body, 0)
    wait_writes(accb0, swb0)
    wait_writes(accb1, swb1)


@jax.jit
def _sc_bond(bond_rm, bt_idx):
    mesh = plsc.VectorSubcoreMesh(core_axis_name="c", subcore_axis_name="s")
    f = pl.kernel(
        _sc_bond_kernel,
        out_type=jax.ShapeDtypeStruct((N, 128), jnp.float32),
        mesh=mesh,
        scratch_types=[
            pltpu.VMEM((DEG, TMAX * C), jnp.int32),
            pltpu.VMEM((TMAX, 4 * C), jnp.int32),
            pltpu.VMEM((4 * C, D_EDGE), jnp.float32),
            pltpu.VMEM((4 * C, D_EDGE), jnp.float32),
            pltpu.VMEM((C, D_EDGE), jnp.float32),
            pltpu.VMEM((C, D_EDGE), jnp.float32),
        ] + [pltpu.SemaphoreType.DMA] * 4,
        compiler_params=pltpu.CompilerParams(use_tc_tiling_on_sc=False),
    )
    return f(bond_rm, bt_idx)


BN = 4000
TSTEPS = N // BN


def _tc_matmul_kernel(nbr_ref, atom_ref, bond_ref, wn_ref, ws_ref, we_ref,
                      bias_ref, act_ref, stats_ref, sacc):
    i = pl.program_id(0)

    @pl.when(i == 0)
    def _():
        sacc[...] = jnp.zeros_like(sacc)

    dn = (((1,), (1,)), ((), ()))
    bf = jnp.bfloat16
    atom = atom_ref[...]
    t = (nbr_ref[...] + atom).astype(bf)
    bond = bond_ref[:, :D_EDGE]
    act = lax.dot_general(t, wn_ref[...].astype(bf), dn,
                          preferred_element_type=jnp.float32)
    act += lax.dot_general(atom.astype(bf), ws_ref[...].astype(bf), dn,
                           preferred_element_type=jnp.float32)
    act += lax.dot_general(bond.astype(bf), we_ref[...].astype(bf), dn,
                           preferred_element_type=jnp.float32)
    act += bias_ref[...]
    act_ref[...] = act.astype(bf)
    sacc[0:1, :] += jnp.sum(act, axis=0, keepdims=True)
    sacc[1:2, :] += jnp.sum(act * act, axis=0, keepdims=True)
    stats_ref[...] = sacc[...]


@jax.jit
def _tc_matmul(nbr_sum, atom_repr, bond_nb, wn, ws, we, bias):
    return pl.pallas_call(
        _tc_matmul_kernel,
        grid=(TSTEPS,),
        in_specs=[
            pl.BlockSpec((BN, D_NODE), lambda i: (i, 0)),
            pl.BlockSpec((BN, D_NODE), lambda i: (i, 0)),
            pl.BlockSpec((BN, 128), lambda i: (i, 0)),
            pl.BlockSpec((D_OUT, D_NODE), lambda i: (0, 0)),
            pl.BlockSpec((D_OUT, D_NODE), lambda i: (0, 0)),
            pl.BlockSpec((D_OUT, D_EDGE), lambda i: (0, 0)),
            pl.BlockSpec((1, D_OUT), lambda i: (0, 0)),
        ],
        out_specs=[
            pl.BlockSpec((BN, D_OUT), lambda i: (i, 0)),
            pl.BlockSpec((8, D_OUT), lambda i: (0, 0)),
        ],
        out_shape=[
            jax.ShapeDtypeStruct((N, D_OUT), jnp.bfloat16),
            jax.ShapeDtypeStruct((8, D_OUT), jnp.float32),
        ],
        scratch_shapes=[pltpu.VMEM((8, D_OUT), jnp.float32)],
    )(nbr_sum, atom_repr, bond_nb, wn, ws, we, bias)


def _tc_norm_kernel(act_ref, stats_ref, out_ref):
    inv_n = jnp.float32(1.0 / N)
    mean = stats_ref[0:1, :] * inv_n
    ex2 = stats_ref[1:2, :] * inv_n
    var = ex2 - mean * mean
    inv = lax.rsqrt(var + jnp.float32(1e-5))
    a = act_ref[...].astype(jnp.float32)
    out_ref[...] = jnp.maximum((a - mean) * inv, 0.0)


@jax.jit
def _tc_norm(act, stats):
    return pl.pallas_call(
        _tc_norm_kernel,
        grid=(TSTEPS,),
        in_specs=[
            pl.BlockSpec((BN, D_OUT), lambda i: (i, 0)),
            pl.BlockSpec((8, D_OUT), lambda i: (0, 0)),
        ],
        out_specs=pl.BlockSpec((BN, D_OUT), lambda i: (i, 0)),
        out_shape=jax.ShapeDtypeStruct((N, D_OUT), jnp.float32),
    )(act, stats)


def kernel(atom_repr, bond_repr, atom_neighbor_idxs, bond_neighbor_idxs,
           W_self, W_deg, bias):
    at_idx = jnp.pad(atom_neighbor_idxs.T, ((0, 0), (0, NPAD - N)))
    bt_idx = jnp.pad(bond_neighbor_idxs.T, ((0, 0), (0, NPAD - N)))
    wn = W_deg[:, :D_NODE]
    we = W_deg[:, D_NODE:]

    nbr_sum = _sc_atom(atom_repr, at_idx)
    bond2 = _prep_bond(bond_repr.T)
    bond_rm = bond2.reshape(EPAD, D_EDGE)
    bond_nb = _sc_bond(bond_rm, bt_idx)
    act, stats = _tc_matmul(nbr_sum, atom_repr, bond_nb, wn, W_self, we, bias)
    return _tc_norm(act, stats)


# speedup vs baseline: 1.2054x; 1.2054x over previous
import jax
import jax.numpy as jnp
import numpy as np
from jax import lax
from jax.experimental import pallas as pl
from jax.experimental.pallas import tpu as pltpu
from jax.experimental.pallas import tpu_sc as plsc

N = 100000
E = 200000
D_NODE = 128
D_EDGE = 16
D_OUT = 128
DEG = 4

NC = 2
NS = 16
NW = NC * NS

C = 32
NCHUNK = N // C
TMAX = (NCHUNK + NW - 1) // NW
NREM = NCHUNK - (TMAX - 1) * NW
NPAD = NW * TMAX * C

EBN = 4096
ESTEPS = (E + EBN - 1) // EBN
EPAD = ESTEPS * EBN

_U_PLACE = np.zeros((D_EDGE, 8, 128), np.float32)
_U_PLACE[np.arange(D_EDGE)[:, None], np.arange(8)[None, :],
         16 * np.arange(8)[None, :] + np.arange(D_EDGE)[:, None]] = 1.0


def _prep_bond_kernel(btab_ref, u_ref, bond2_ref):
    del u_ref
    y = jnp.transpose(btab_ref[...])
    y3 = y.reshape(EBN // 8, 8, D_EDGE)
    bond2_ref[...] = jnp.concatenate([y3[:, pp, :] for pp in range(8)],
                                     axis=1)


@jax.jit
def _prep_bond(btab):
    return pl.pallas_call(
        _prep_bond_kernel,
        grid=(ESTEPS,),
        in_specs=[
            pl.BlockSpec((D_EDGE, EBN), lambda i: (0, i)),
            pl.BlockSpec((D_EDGE, 8, 128), lambda i: (0, 0, 0)),
        ],
        out_specs=pl.BlockSpec((EBN // 8, 128), lambda i: (i, 0)),
        out_shape=jax.ShapeDtypeStruct((EPAD // 8, 128), jnp.float32),
    )(btab, _U_PLACE)


def _sc_gather_kernel(atom_hbm, bond_hbm, at_hbm, bt_hbm,
                      nbr_out, bnd_out,
                      raw_i, idx_all, bufa0, bufa1, bufb0, bufb1,
                      acca0, acca1, accb0, accb1,
                      sga0, sgb0, sga1, sgb1, swa0, swb0, swa1, swb1):
    wid = lax.axis_index("c") * NS + lax.axis_index("s")
    start_c = wid * (TMAX - 1) + jnp.minimum(wid, NREM)
    nt = jnp.where(wid < NREM, TMAX, TMAX - 1)
    a0 = start_c * C

    pltpu.sync_copy(at_hbm.at[pl.ds(1, DEG), pl.ds(a0, TMAX * C)],
                    raw_i.at[0])
    pltpu.sync_copy(bt_hbm.at[:, pl.ds(a0, TMAX * C)], raw_i.at[1])

    def build(t, _):
        for k in range(DEG):
            for h in range(C // 16):
                dst = pl.ds(C * k + 16 * h, 16)
                src = pl.ds(C * t + 16 * h, 16)
                idx_all[t, 0, dst] = raw_i[0, k, src]
                idx_all[t, 1, dst] = raw_i[1, k, src]
        return 0

    lax.fori_loop(0, TMAX, build, 0)

    def issue_gather(t, bufa, bufb, sa, sb):
        pltpu.async_copy(atom_hbm.at[idx_all.at[t, 0]], bufa, sa)
        pltpu.async_copy(bond_hbm.at[idx_all.at[t, 1]], bufb, sb)

    def wait_gather(t, bufa, bufb, sa, sb):
        pltpu.make_async_copy(atom_hbm.at[idx_all.at[t, 0]], bufa, sa).wait()
        pltpu.make_async_copy(bond_hbm.at[idx_all.at[t, 1]], bufb, sb).wait()

    def do_sums(bufa, bufb, acca, accb):
        @plsc.parallel_loop(0, C, 1, unroll=4)
        def _(r):
            for d in range(D_NODE // 16):
                sl = pl.ds(16 * d, 16)
                acca[r, sl] = (bufa[r, sl] + bufa[C + r, sl]
                               + bufa[2 * C + r, sl] + bufa[3 * C + r, sl])
            accb[r, :] = (bufb[r, :] + bufb[C + r, :]
                          + bufb[2 * C + r, :] + bufb[3 * C + r, :])

    def issue_writes(t, acca, accb, sa, sb):
        obase = (start_c + t) * C
        pltpu.async_copy(acca, nbr_out.at[pl.ds(obase, C)], sa)
        pltpu.async_copy(accb, bnd_out.at[pl.ds(obase, C), pl.ds(0, D_EDGE)],
                         sb)

    def wait_writes(acca, accb, sa, sb):
        pltpu.make_async_copy(acca, nbr_out.at[pl.ds(0, C)], sa).wait()
        pltpu.make_async_copy(accb, bnd_out.at[pl.ds(0, C), pl.ds(0, D_EDGE)],
                              sb).wait()

    issue_gather(0, bufa0, bufb0, sga0, sgb0)

    def body(u, _):
        t0 = 2 * u
        t1 = t0 + 1

        @pl.when(t1 < nt)
        def _():
            issue_gather(t1, bufa1, bufb1, sga1, sgb1)

        wait_gather(t0, bufa0, bufb0, sga0, sgb0)

        @pl.when(u >= 1)
        def _():
            wait_writes(acca0, accb0, swa0, swb0)

        do_sums(bufa0, bufb0, acca0, accb0)
        issue_writes(t0, acca0, accb0, swa0, swb0)

        @pl.when(t1 < nt)
        def _():
            @pl.when(t1 + 1 < nt)
            def _():
                issue_gather(t1 + 1, bufa0, bufb0, sga0, sgb0)

            wait_gather(t1, bufa1, bufb1, sga1, sgb1)

            @pl.when(u >= 1)
            def _():
                wait_writes(acca1, accb1, swa1, swb1)

            do_sums(bufa1, bufb1, acca1, accb1)
            issue_writes(t1, acca1, accb1, swa1, swb1)

        return 0

    lax.fori_loop(0, (nt + 1) // 2, body, 0)

    wait_writes(acca0, accb0, swa0, swb0)
    wait_writes(acca1, accb1, swa1, swb1)


@jax.jit
def _sc_gather(atom_repr, bond_rm, at_idx, bt_idx):
    mesh = plsc.VectorSubcoreMesh(core_axis_name="c", subcore_axis_name="s")
    f = pl.kernel(
        _sc_gather_kernel,
        out_type=[
            jax.ShapeDtypeStruct((N, D_NODE), jnp.float32),
            jax.ShapeDtypeStruct((N, 128), jnp.float32),
        ],
        mesh=mesh,
        scratch_types=[
            pltpu.VMEM((2, DEG, TMAX * C), jnp.int32),
            pltpu.VMEM((TMAX, 2, 4 * C), jnp.int32),
            pltpu.VMEM((4 * C, D_NODE), jnp.float32),
            pltpu.VMEM((4 * C, D_NODE), jnp.float32),
            pltpu.VMEM((4 * C, D_EDGE), jnp.float32),
            pltpu.VMEM((4 * C, D_EDGE), jnp.float32),
            pltpu.VMEM((C, D_NODE), jnp.float32),
            pltpu.VMEM((C, D_NODE), jnp.float32),
            pltpu.VMEM((C, D_EDGE), jnp.float32),
            pltpu.VMEM((C, D_EDGE), jnp.float32),
        ] + [pltpu.SemaphoreType.DMA] * 8,
        compiler_params=pltpu.CompilerParams(use_tc_tiling_on_sc=False),
    )
    return f(atom_repr, bond_rm, at_idx, bt_idx)


BN = 4000
TSTEPS = N // BN


def _tc_matmul_kernel(nbr_ref, atom_ref, bond_ref, wn_ref, ws_ref, we_ref,
                      bias_ref, act_ref, stats_ref, sacc):
    i = pl.program_id(0)

    @pl.when(i == 0)
    def _():
        sacc[...] = jnp.zeros_like(sacc)

    dn = (((1,), (1,)), ((), ()))
    bf = jnp.bfloat16
    atom = atom_ref[...]
    t = (nbr_ref[...] + atom).astype(bf)
    bond = bond_ref[:, :D_EDGE]
    act = lax.dot_general(t, wn_ref[...].astype(bf), dn,
                          preferred_element_type=jnp.float32)
    act += lax.dot_general(atom.astype(bf), ws_ref[...].astype(bf), dn,
                           preferred_element_type=jnp.float32)
    act += lax.dot_general(bond.astype(bf), we_ref[...].astype(bf), dn,
                           preferred_element_type=jnp.float32)
    act += bias_ref[...]
    act_ref[...] = act.astype(bf)
    sacc[0:1, :] += jnp.sum(act, axis=0, keepdims=True)
    sacc[1:2, :] += jnp.sum(act * act, axis=0, keepdims=True)
    stats_ref[...] = sacc[...]


@jax.jit
def _tc_matmul(nbr_sum, atom_repr, bond_nb, wn, ws, we, bias):
    return pl.pallas_call(
        _tc_matmul_kernel,
        grid=(TSTEPS,),
        in_specs=[
            pl.BlockSpec((BN, D_NODE), lambda i: (i, 0)),
            pl.BlockSpec((BN, D_NODE), lambda i: (i, 0)),
            pl.BlockSpec((BN, 128), lambda i: (i, 0)),
            pl.BlockSpec((D_OUT, D_NODE), lambda i: (0, 0)),
            pl.BlockSpec((D_OUT, D_NODE), lambda i: (0, 0)),
            pl.BlockSpec((D_OUT, D_EDGE), lambda i: (0, 0)),
            pl.BlockSpec((1, D_OUT), lambda i: (0, 0)),
        ],
        out_specs=[
            pl.BlockSpec((BN, D_OUT), lambda i: (i, 0)),
            pl.BlockSpec((8, D_OUT), lambda i: (0, 0)),
        ],
        out_shape=[
            jax.ShapeDtypeStruct((N, D_OUT), jnp.bfloat16),
            jax.ShapeDtypeStruct((8, D_OUT), jnp.float32),
        ],
        scratch_shapes=[pltpu.VMEM((8, D_OUT), jnp.float32)],
    )(nbr_sum, atom_repr, bond_nb, wn, ws, we, bias)


def _tc_norm_kernel(act_ref, stats_ref, out_ref):
    inv_n = jnp.float32(1.0 / N)
    mean = stats_ref[0:1, :] * inv_n
    ex2 = stats_ref[1:2, :] * inv_n
    var = ex2 - mean * mean
    inv = lax.rsqrt(var + jnp.float32(1e-5))
    a = act_ref[...].astype(jnp.float32)
    out_ref[...] = jnp.maximum((a - mean) * inv, 0.0)


@jax.jit
def _tc_norm(act, stats):
    return pl.pallas_call(
        _tc_norm_kernel,
        grid=(TSTEPS,),
        in_specs=[
            pl.BlockSpec((BN, D_OUT), lambda i: (i, 0)),
            pl.BlockSpec((8, D_OUT), lambda i: (0, 0)),
        ],
        out_specs=pl.BlockSpec((BN, D_OUT), lambda i: (i, 0)),
        out_shape=jax.ShapeDtypeStruct((N, D_OUT), jnp.float32),
    )(act, stats)


def kernel(atom_repr, bond_repr, atom_neighbor_idxs, bond_neighbor_idxs,
           W_self, W_deg, bias):
    at_idx = jnp.pad(atom_neighbor_idxs.T, ((0, 0), (0, NPAD - N)))
    bt_idx = jnp.pad(bond_neighbor_idxs.T, ((0, 0), (0, NPAD - N)))
    wn = W_deg[:, :D_NODE]
    we = W_deg[:, D_NODE:]

    bond2 = _prep_bond(bond_repr.T)
    bond_rm = bond2.reshape(EPAD, D_EDGE)
    nbr_sum, bond_nb = _sc_gather(atom_repr, bond_rm, at_idx, bt_idx)
    act, stats = _tc_matmul(nbr_sum, atom_repr, bond_nb, wn, W_self, we, bias)
    return _tc_norm(act, stats)
